# baseline (speedup 1.0000x reference)
import functools

import jax
import jax.numpy as jnp
from jax import lax
from jax.experimental import pallas as pl
from jax.experimental.pallas import tpu as pltpu
from jax.experimental.pallas import tpu_sc as plsc

BN = 512
LANE = 128
IDX_BITS = 0x7FF
IMAX = 2147483647

try:
    _SC_INFO = plsc.get_sparse_core_info()
    _NC = _SC_INFO.num_cores
    _NS = _SC_INFO.num_subcores
except Exception:
    _NC, _NS = 2, 16
NW = _NC * _NS
CH = 64


def _top3_body(xyz1_ref, xyz2_ref, w_ref, kb_ref, s1_ref, s2_ref, s3_ref):
    x = xyz1_ref[0]
    y = xyz2_ref[0]
    n2 = y.shape[0]
    nchunk = n2 // LANE
    cbits = nchunk - 1
    rnd = cbits // 2 + 1

    xx = jnp.sum(x * x, axis=1)
    yy = jnp.sum(y * y, axis=1)
    xy = lax.dot_general(x, y, (((1,), (1,)), ((), ())),
                         preferred_element_type=jnp.float32)
    dists = xx[:, None] + yy[None, :] - 2.0 * xy

    db = lax.bitcast_convert_type(dists, jnp.int32)
    dbm = jnp.where(db < 0, db ^ 0x7FFFFFFF, db)
    kb_ref[...] = (dbm + rnd) & ~cbits

    s1_ref[...] = kb_ref[:, 0:LANE]
    s2_ref[...] = jnp.full((x.shape[0], LANE), IMAX, jnp.int32)
    s3_ref[...] = jnp.full((x.shape[0], LANE), IMAX, jnp.int32)

    def cbody(c, carry):
        kc = kb_ref[:, pl.ds(c * LANE, LANE)] | c
        s1 = s1_ref[...]
        s2 = s2_ref[...]
        s3 = s3_ref[...]
        t = jnp.maximum(s1, kc)
        s1_ref[...] = jnp.minimum(s1, kc)
        t2 = jnp.maximum(s2, t)
        s2_ref[...] = jnp.minimum(s2, t)
        s3_ref[...] = jnp.minimum(s3, t2)
        return carry

    lax.fori_loop(1, nchunk, cbody, 0)

    s = jnp.concatenate([s1_ref[...], s2_ref[...], s3_ref[...]], axis=1)
    pio = lax.broadcasted_iota(jnp.int32, s.shape, 1) & (LANE - 1)
    ks, ps = [], []
    for _ in range(3):
        kk = jnp.min(s, axis=1)
        m = s == kk[:, None]
        ps.append(jnp.min(jnp.where(m, pio, LANE), axis=1))
        s = jnp.where(m, IMAX, s)
        ks.append(kk)

    recips = []
    gidx = []
    for kk, pp in zip(ks, ps):
        v = kk & ~cbits
        dk = lax.bitcast_convert_type(
            jnp.where(v < 0, v ^ 0x7FFFFFFF, v), jnp.float32)
        recips.append(1.0 / (dk + 1e-8))
        gidx.append((kk & cbits) * LANE + pp)
    norm = recips[0] + recips[1] + recips[2]

    lane = lax.broadcasted_iota(jnp.int32, (x.shape[0], LANE), 1)
    lk = lane >> 4
    ws = [(r / norm)[:, None] for r in recips]
    wpart = jnp.where(
        lk == 0, ws[0], jnp.where(lk == 1, ws[1],
                                  jnp.where(lk == 2, ws[2], 0.0)))
    gf = [g.astype(jnp.float32)[:, None] for g in gidx]
    out = jnp.where(lane == 64, gf[0],
                    jnp.where(lane == 65, gf[1],
                              jnp.where(lane == 66, gf[2], wpart)))
    w_ref[0] = out


def _sc_interp(f2_flat, idx_flat, w_flat, n_queries, cin):
    nq = n_queries
    qpw = nq // NW
    n_chunks = qpw // CH
    mesh = plsc.VectorSubcoreMesh(core_axis_name="c", subcore_axis_name="s")

    @functools.partial(
        pl.kernel, mesh=mesh,
        out_type=jax.ShapeDtypeStruct((nq, cin), jnp.float32),
        scratch_types=[
            pltpu.VMEM((CH,), jnp.int32),
            pltpu.VMEM((CH,), jnp.int32),
            pltpu.VMEM((CH,), jnp.int32),
            pltpu.VMEM((CH * 16,), jnp.float32),
            pltpu.VMEM((CH * 16,), jnp.float32),
            pltpu.VMEM((CH * 16,), jnp.float32),
            pltpu.VMEM((CH, cin), jnp.float32),
            pltpu.VMEM((CH, cin), jnp.float32),
            pltpu.VMEM((CH, cin), jnp.float32),
            pltpu.VMEM((CH, cin), jnp.float32),
            pltpu.SemaphoreType.DMA,
        ],
    )
    def body(f2_hbm, idx_hbm, w_hbm, out_hbm, i0, i1, i2, w0, w1, w2,
             r0, r1, r2, out_v, sem):
        wid = lax.axis_index("s") * _NC + lax.axis_index("c")
        idxs = (i0, i1, i2)
        wrefs = (w0, w1, w2)
        rows = (r0, r1, r2)

        def chunk(ci, carry):
            qb = wid * qpw + ci * CH
            for k in range(3):
                pltpu.sync_copy(idx_hbm.at[pl.ds(k * nq + qb, CH)], idxs[k])
                pltpu.sync_copy(w_hbm.at[pl.ds((k * nq + qb) * 16, CH * 16)],
                                wrefs[k])
            cps = [pltpu.async_copy(f2_hbm.at[idxs[k]], rows[k], sem)
                   for k in range(3)]
            for cp in cps:
                cp.wait()

            def q_body(q, c2):
                wsl = pl.ds(q * 16, 16)
                ws = [w0[wsl], w1[wsl], w2[wsl]]
                for c in range(cin // 16):
                    sl = pl.ds(c * 16, 16)
                    out_v[q, sl] = (ws[0] * r0[q, sl] + ws[1] * r1[q, sl]
                                    + ws[2] * r2[q, sl])
                return c2

            lax.fori_loop(0, CH, q_body, 0)
            pltpu.sync_copy(out_v, out_hbm.at[pl.ds(qb, CH)])
            return carry

        lax.fori_loop(0, n_chunks, chunk, 0)

    return body(f2_flat, idx_flat, w_flat)


def _mlp_body(f1_ref, interp_ref, w1a_ref, w1b_ref, b1_ref, gamma_ref,
              beta_ref, w2_ref, b2_ref, out_ref):
    h = (lax.dot_general(f1_ref[0], w1a_ref[...], (((1,), (0,)), ((), ())),
                         preferred_element_type=jnp.float32)
         + lax.dot_general(interp_ref[0], w1b_ref[...],
                           (((1,), (0,)), ((), ())),
                           preferred_element_type=jnp.float32)
         + b1_ref[...])
    mu = jnp.mean(h, axis=1, keepdims=True)
    xc = h - mu
    var = jnp.mean(xc * xc, axis=1, keepdims=True)
    h = xc * lax.rsqrt(var + 1e-5) * gamma_ref[...] + beta_ref[...]
    h = jnp.maximum(h, 0.0)
    out_ref[0] = (lax.dot_general(h, w2_ref[...], (((1,), (0,)), ((), ())),
                                  preferred_element_type=jnp.float32)
                  + b2_ref[...])


@jax.jit
def kernel(xyz1, xyz2, features1, features2, W1, b1, gamma, beta, W2, b2):
    B, N1, _ = xyz1.shape
    _, N2, _ = xyz2.shape
    Cskip = features1.shape[-1]
    Cin = features2.shape[-1]
    Cout = W2.shape[-1]
    nq = B * N1
    nblk = N1 // BN

    w_all = pl.pallas_call(
        _top3_body,
        grid=(B, nblk),
        in_specs=[
            pl.BlockSpec((1, BN, 3), lambda b, i: (b, i, 0)),
            pl.BlockSpec((1, N2, 3), lambda b, i: (b, 0, 0)),
        ],
        out_specs=pl.BlockSpec((1, BN, 128), lambda b, i: (b, i, 0)),
        out_shape=jax.ShapeDtypeStruct((B, N1, 128), jnp.float32),
        scratch_shapes=[
            pltpu.VMEM((BN, N2), jnp.int32),
            pltpu.VMEM((BN, 128), jnp.int32),
            pltpu.VMEM((BN, 128), jnp.int32),
            pltpu.VMEM((BN, 128), jnp.int32),
        ],
    )(xyz1, xyz2)
    w_all = w_all.reshape(nq, 128)
    boffs = (jnp.arange(nq, dtype=jnp.int32) // N1 * N2)[:, None]
    idx_all = w_all[:, 64:67].astype(jnp.int32) + boffs

    f2_flat = features2.reshape(B * N2, Cin)
    idx_flat = idx_all.T.reshape(-1)
    w_flat = (w_all[:, :48].reshape(nq, 3, 16).transpose(1, 0, 2).reshape(-1))
    interp = _sc_interp(f2_flat, idx_flat, w_flat,
                        nq, Cin).reshape(B, N1, Cin)
    W1a = W1[:Cskip]
    W1b = W1[Cskip:]
    const = lambda shape: pl.BlockSpec(shape, lambda b, i: (0,) * len(shape))
    out = pl.pallas_call(
        _mlp_body,
        grid=(B, nblk),
        in_specs=[
            pl.BlockSpec((1, BN, Cskip), lambda b, i: (b, i, 0)),
            pl.BlockSpec((1, BN, Cin), lambda b, i: (b, i, 0)),
            const((Cskip, Cout)),
            const((Cin, Cout)),
            const((1, Cout)),
            const((1, Cout)),
            const((1, Cout)),
            const((Cout, Cout)),
            const((1, Cout)),
        ],
        out_specs=pl.BlockSpec((1, BN, Cout), lambda b, i: (b, i, 0)),
        out_shape=jax.ShapeDtypeStruct((B, N1, Cout), jnp.float32),
    )(features1, interp, W1a, W1b, b1.reshape(1, -1), gamma.reshape(1, -1),
      beta.reshape(1, -1), W2, b2.reshape(1, -1))
    return out
